# baseline (speedup 1.0000x reference)
import jax
import jax.numpy as jnp
from jax import lax
from jax.experimental import pallas as pl
from jax.experimental.pallas import tpu as pltpu
from jax.experimental.pallas import tpu_sc as plsc

_NC = 2
_NS = 16
_L = 16
_NW = _NC * _NS

_B, _K, _D = 16384, 32, 64
_BPW = _B // _NW
_NBT = _BPW // 128
_NDT = _D // 8
_M = _NDT * (_B // 128)


def _sc_body(pi_hbm, g_hbm, sg_hbm,
             osg, idx_out,
             pi_v, g_v, idx_v, slab0, slab1, tile_v, sem0, sem1):
  wid = lax.axis_index("s") * _NC + lax.axis_index("c")
  base = wid * _BPW
  bt0 = wid * _NBT

  def slab_src(ref, it):
    bt = bt0 + it // _NDT
    dt = it % _NDT
    return ref.at[:, dt * (_B // 128) + bt]

  slabs = (slab0, slab1)
  sems = (sem0, sem1)

  pltpu.async_copy(slab_src(sg_hbm, 0), slab0, sem0)
  pltpu.async_copy(slab_src(sg_hbm, 1), slab1, sem1)

  pltpu.sync_copy(pi_hbm.at[:, pl.ds(base, _BPW)], pi_v)
  pltpu.sync_copy(g_hbm.at[:, pl.ds(base, _BPW)], g_v)

  def argmax_group(j, _):
    sl = pl.ds(j * _L, _L)
    best_v = pi_v[0, sl] + g_v[0, sl]
    best_i = jnp.zeros((_L,), jnp.int32)
    for k in range(1, _K):
      v = pi_v[k, sl] + g_v[k, sl]
      p = v > best_v
      best_v = jnp.where(p, v, best_v)
      best_i = jnp.where(p, jnp.int32(k), best_i)
    idx_v[sl] = best_i
    return 0

  lax.fori_loop(0, _BPW // _L, argmax_group, 0, unroll=False)
  pltpu.sync_copy(idx_v, idx_out.at[pl.ds(base, _BPW)])

  lanes = lax.iota(jnp.int32, _L)
  nslab = _NBT * _NDT

  def two_slabs(t, _):
    for j in range(2):
      it = t * 2 + j
      buf = slabs[j]
      sem = sems[j]
      pltpu.make_async_copy(slab_src(sg_hbm, it), buf, sem).wait()

      lo = (it // _NDT) * 128
      for lg in range(8):
        k16 = idx_v[pl.ds(lo + lg * _L, _L)]
        l16 = lanes + lg * _L
        for ds_ in range(8):
          d16 = jnp.full((_L,), ds_, jnp.int32)
          tile_v[ds_, pl.ds(lg * _L, _L)] = plsc.load_gather(
              buf, [k16, d16, l16])

      bt = bt0 + it // _NDT
      dt = it % _NDT
      pltpu.sync_copy(
          tile_v, osg.at[pl.ds(dt * 8, 8), pl.ds(bt * 128, 128)])

      nit = it + 2
      @pl.when(nit < nslab)
      def _():
        pltpu.async_copy(slab_src(sg_hbm, nit), buf, sem)
    return 0

  lax.fori_loop(0, nslab // 2, two_slabs, 0, unroll=False)


def _tc_body(pi_ref, g_ref, mu_ref, omu_ref, bidx_ref):
  dt = pl.program_id(1)

  @pl.when(dt == 0)
  def _():
    z = pi_ref[...] + g_ref[...]
    best_v = z[0:1, :]
    best_i = jnp.zeros((1, 128), jnp.int32)
    for k in range(1, _K):
      v = z[k:k + 1, :]
      p = v > best_v
      best_v = jnp.where(p, v, best_v)
      best_i = jnp.where(p, jnp.int32(k), best_i)
    bidx_ref[...] = best_i

  bidx = bidx_ref[...]
  acc = mu_ref[0, 0]
  for k in range(1, _K):
    acc = jnp.where(bidx == jnp.int32(k), mu_ref[k, 0], acc)
  omu_ref[...] = acc


@jax.jit
def kernel(pi, mu, sigma):
  B, K = pi.shape
  D = mu.shape[2]
  g = jax.random.gumbel(jax.random.key(42), (B, K), pi.dtype)

  piT = pi.T
  gT = g.T
  mu5 = mu.transpose(1, 2, 0).reshape(K, _NDT, 8, B // 128, 128)
  mu5 = mu5.transpose(0, 1, 3, 2, 4).reshape(K, _M, 8, 128)
  sg5 = sigma.transpose(1, 2, 0).reshape(K, _NDT, 8, B // 128, 128)
  sg5 = sg5.transpose(0, 1, 3, 2, 4).reshape(K, _M, 8, 128)

  omu = pl.pallas_call(
      _tc_body,
      grid=(B // 128, _NDT),
      in_specs=[
          pl.BlockSpec((K, 128), lambda bt, dt: (0, bt)),
          pl.BlockSpec((K, 128), lambda bt, dt: (0, bt)),
          pl.BlockSpec((K, 1, 8, 128), lambda bt, dt: (0, dt * 128 + bt, 0, 0)),
      ],
      out_specs=pl.BlockSpec((8, 128), lambda bt, dt: (dt, bt)),
      out_shape=jax.ShapeDtypeStruct((D, B), jnp.float32),
      scratch_shapes=[pltpu.VMEM((1, 128), jnp.int32)],
      compiler_params=pltpu.CompilerParams(
          dimension_semantics=("arbitrary", "arbitrary")),
  )(piT, gT, mu5)

  mesh = plsc.VectorSubcoreMesh(core_axis_name="c", subcore_axis_name="s")
  run = pl.kernel(
      _sc_body,
      out_type=(
          jax.ShapeDtypeStruct((D, B), jnp.float32),
          jax.ShapeDtypeStruct((B,), jnp.int32),
      ),
      mesh=mesh,
      compiler_params=pltpu.CompilerParams(needs_layout_passes=False),
      scratch_types=[
          pltpu.VMEM((K, _BPW), jnp.float32),
          pltpu.VMEM((K, _BPW), jnp.float32),
          pltpu.VMEM((_BPW,), jnp.int32),
          pltpu.VMEM((K, 8, 128), jnp.float32),
          pltpu.VMEM((K, 8, 128), jnp.float32),
          pltpu.VMEM((8, 128), jnp.float32),
          pltpu.SemaphoreType.DMA,
          pltpu.SemaphoreType.DMA,
      ],
  )
  osg, idx = run(piT, gT, sg5)
  return omu.T, osg.T, idx


# speedup vs baseline: 3.8462x; 3.8462x over previous
import jax
import jax.numpy as jnp
from jax import lax
from jax.experimental import pallas as pl
from jax.experimental.pallas import tpu as pltpu
from jax.experimental.pallas import tpu_sc as plsc

_NC = 2
_NS = 16
_L = 16
_NW = _NC * _NS

_B, _K, _D = 16384, 32, 64
_BPW = _B // _NW
_NBT = _BPW // 128
_NDT = _D // 8
_M = _NDT * (_B // 128)


def _sc_body(pi_hbm, g_hbm, sg_hbm,
             osg, idx_out,
             pi_v, g_v, idx_v, slab0, slab1, tile_v, sem0, sem1):
  wid = lax.axis_index("s") * _NC + lax.axis_index("c")
  base = wid * _BPW
  bt0 = wid * _NBT

  def slab_src(ref, it):
    bt = bt0 + it // _NDT
    dt = it % _NDT
    return ref.at[:, dt * (_B // 128) + bt]

  slabs = (slab0, slab1)
  sems = (sem0, sem1)

  pltpu.async_copy(slab_src(sg_hbm, 0), slab0, sem0)
  pltpu.async_copy(slab_src(sg_hbm, 1), slab1, sem1)

  pltpu.sync_copy(pi_hbm.at[:, pl.ds(base, _BPW)], pi_v)
  pltpu.sync_copy(g_hbm.at[:, pl.ds(base, _BPW)], g_v)

  def argmax_group(j, _):
    sl = pl.ds(j * _L, _L)
    best_v = pi_v[0, sl] + g_v[0, sl]
    best_i = jnp.zeros((_L,), jnp.int32)
    for k in range(1, _K):
      v = pi_v[k, sl] + g_v[k, sl]
      p = v > best_v
      best_v = jnp.where(p, v, best_v)
      best_i = jnp.where(p, jnp.int32(k), best_i)
    idx_v[sl] = best_i
    return 0

  lax.fori_loop(0, _BPW // _L, argmax_group, 0, unroll=False)
  pltpu.sync_copy(idx_v, idx_out.at[pl.ds(base, _BPW)])

  lanes = lax.iota(jnp.int32, _L)
  nslab = _NBT * _NDT

  def two_slabs(t, _):
    for j in range(2):
      it = t * 2 + j
      buf = slabs[j]
      sem = sems[j]
      pltpu.make_async_copy(slab_src(sg_hbm, it), buf, sem).wait()

      lo = (it // _NDT) * 128
      for lg in range(8):
        k16 = idx_v[pl.ds(lo + lg * _L, _L)]
        l16 = lanes + lg * _L
        for ds_ in range(8):
          d16 = jnp.full((_L,), ds_, jnp.int32)
          tile_v[ds_, pl.ds(lg * _L, _L)] = plsc.load_gather(
              buf, [k16, d16, l16])

      bt = bt0 + it // _NDT
      dt = it % _NDT
      pltpu.sync_copy(
          tile_v, osg.at[pl.ds(dt * 8, 8), pl.ds(bt * 128, 128)])

      nit = it + 2
      @pl.when(nit < nslab)
      def _():
        pltpu.async_copy(slab_src(sg_hbm, nit), buf, sem)
    return 0

  lax.fori_loop(0, nslab // 2, two_slabs, 0, unroll=False)


_MB = 8
_TCW = _MB * 128


def _tc_body(pi_ref, g_ref, mu_ref, omu_ref, bidx_ref):
  dt = pl.program_id(1)

  @pl.when(dt == 0)
  def _():
    z = pi_ref[...] + g_ref[...]
    best_v = z[0:1, :]
    best_i = jnp.zeros((1, _TCW), jnp.int32)
    for k in range(1, _K):
      v = z[k:k + 1, :]
      p = v > best_v
      best_v = jnp.where(p, v, best_v)
      best_i = jnp.where(p, jnp.int32(k), best_i)
    bidx_ref[...] = best_i

  for i in range(_MB):
    bidx = bidx_ref[:, pl.ds(i * 128, 128)]
    acc = mu_ref[0, i]
    for k in range(1, _K):
      acc = jnp.where(bidx == jnp.int32(k), mu_ref[k, i], acc)
    omu_ref[:, pl.ds(i * 128, 128)] = acc


@jax.jit
def kernel(pi, mu, sigma):
  B, K = pi.shape
  D = mu.shape[2]
  g = jax.random.gumbel(jax.random.key(42), (B, K), pi.dtype)

  piT = pi.T
  gT = g.T
  mu5 = mu.transpose(1, 2, 0).reshape(K, _NDT, 8, B // 128, 128)
  mu5 = mu5.transpose(0, 1, 3, 2, 4).reshape(K, _M, 8, 128)
  sg5 = sigma.transpose(1, 2, 0).reshape(K, _NDT, 8, B // 128, 128)
  sg5 = sg5.transpose(0, 1, 3, 2, 4).reshape(K, _M, 8, 128)

  omu = pl.pallas_call(
      _tc_body,
      grid=(B // 128 // _MB, _NDT),
      in_specs=[
          pl.BlockSpec((K, _TCW), lambda bt, dt: (0, bt)),
          pl.BlockSpec((K, _TCW), lambda bt, dt: (0, bt)),
          pl.BlockSpec((K, _MB, 8, 128),
                       lambda bt, dt: (0, dt * (_B // 128 // _MB) + bt, 0, 0)),
      ],
      out_specs=pl.BlockSpec((8, _TCW), lambda bt, dt: (dt, bt)),
      out_shape=jax.ShapeDtypeStruct((D, B), jnp.float32),
      scratch_shapes=[pltpu.VMEM((1, _TCW), jnp.int32)],
      compiler_params=pltpu.CompilerParams(
          dimension_semantics=("arbitrary", "arbitrary")),
  )(piT, gT, mu5)

  mesh = plsc.VectorSubcoreMesh(core_axis_name="c", subcore_axis_name="s")
  run = pl.kernel(
      _sc_body,
      out_type=(
          jax.ShapeDtypeStruct((D, B), jnp.float32),
          jax.ShapeDtypeStruct((B,), jnp.int32),
      ),
      mesh=mesh,
      compiler_params=pltpu.CompilerParams(needs_layout_passes=False),
      scratch_types=[
          pltpu.VMEM((K, _BPW), jnp.float32),
          pltpu.VMEM((K, _BPW), jnp.float32),
          pltpu.VMEM((_BPW,), jnp.int32),
          pltpu.VMEM((K, 8, 128), jnp.float32),
          pltpu.VMEM((K, 8, 128), jnp.float32),
          pltpu.VMEM((8, 128), jnp.float32),
          pltpu.SemaphoreType.DMA,
          pltpu.SemaphoreType.DMA,
      ],
  )
  osg, idx = run(piT, gT, sg5)
  return omu.T, osg.T, idx


# speedup vs baseline: 4.7082x; 1.2241x over previous
import jax
import jax.numpy as jnp
from jax import lax
from jax.experimental import pallas as pl
from jax.experimental.pallas import tpu as pltpu
from jax.experimental.pallas import tpu_sc as plsc

_NC = 2
_NS = 16
_L = 16
_NW = _NC * _NS

_B, _K, _D = 16384, 32, 64
_BPW = _B // _NW
_NBT = _BPW // 128
_NDT = _D // 8
_M = _NDT * (_B // 128)


def _sc_body(pi_hbm, g_hbm, sg_hbm,
             osg, idx_out,
             pi_v, g_v, idx_v, slab0, slab1, tile_v, sem0, sem1):
  wid = lax.axis_index("s") * _NC + lax.axis_index("c")
  base = wid * _BPW
  bt0 = wid * _NBT

  def slab_src(ref, it):
    bt = bt0 + it // _NDT
    dt = it % _NDT
    return ref.at[:, dt * (_B // 128) + bt]

  slabs = (slab0, slab1)
  sems = (sem0, sem1)

  pltpu.async_copy(slab_src(sg_hbm, 0), slab0, sem0)
  pltpu.async_copy(slab_src(sg_hbm, 1), slab1, sem1)

  pltpu.sync_copy(pi_hbm.at[:, pl.ds(base, _BPW)], pi_v)
  pltpu.sync_copy(g_hbm.at[:, pl.ds(base, _BPW)], g_v)

  def argmax_group(j, _):
    sl = pl.ds(j * _L, _L)
    best_v = pi_v[0, sl] + g_v[0, sl]
    best_i = jnp.zeros((_L,), jnp.int32)
    for k in range(1, _K):
      v = pi_v[k, sl] + g_v[k, sl]
      p = v > best_v
      best_v = jnp.where(p, v, best_v)
      best_i = jnp.where(p, jnp.int32(k), best_i)
    idx_v[sl] = best_i
    return 0

  lax.fori_loop(0, _BPW // _L, argmax_group, 0, unroll=False)
  pltpu.sync_copy(idx_v, idx_out.at[pl.ds(base, _BPW)])

  lanes = lax.iota(jnp.int32, _L)
  nslab = _NBT * _NDT

  def two_slabs(t, _):
    for j in range(2):
      it = t * 2 + j
      buf = slabs[j]
      sem = sems[j]
      pltpu.make_async_copy(slab_src(sg_hbm, it), buf, sem).wait()

      lo = (it // _NDT) * 128
      for lg in range(8):
        k16 = idx_v[pl.ds(lo + lg * _L, _L)]
        l16 = lanes + lg * _L
        for ds_ in range(8):
          d16 = jnp.full((_L,), ds_, jnp.int32)
          tile_v[ds_, pl.ds(lg * _L, _L)] = plsc.load_gather(
              buf, [k16, d16, l16])

      bt = bt0 + it // _NDT
      dt = it % _NDT
      pltpu.sync_copy(
          tile_v, osg.at[pl.ds(dt * 8, 8), pl.ds(bt * 128, 128)])

      nit = it + 2
      @pl.when(nit < nslab)
      def _():
        pltpu.async_copy(slab_src(sg_hbm, nit), buf, sem)
    return 0

  lax.fori_loop(0, nslab // 2, two_slabs, 0, unroll=False)


_MB = 16
_TCW = _MB * 128


def _tc_body(pi_ref, g_ref, mu_ref, omu_ref, bidx_ref):
  dt = pl.program_id(1)

  @pl.when(dt == 0)
  def _():
    z = pi_ref[...] + g_ref[...]
    best_v = z[0:1, :]
    best_i = jnp.zeros((1, _TCW), jnp.int32)
    for k in range(1, _K):
      v = z[k:k + 1, :]
      p = v > best_v
      best_v = jnp.where(p, v, best_v)
      best_i = jnp.where(p, jnp.int32(k), best_i)
    bidx_ref[...] = jnp.broadcast_to(best_i, (8, _TCW))

  for i in range(_MB):
    bidx = bidx_ref[:, pl.ds(i * 128, 128)]
    acc = mu_ref[0, i]
    for k in range(1, _K):
      acc = jnp.where(bidx == jnp.int32(k), mu_ref[k, i], acc)
    omu_ref[:, pl.ds(i * 128, 128)] = acc


@jax.jit
def kernel(pi, mu, sigma):
  B, K = pi.shape
  D = mu.shape[2]
  g = jax.random.gumbel(jax.random.key(42), (B, K), pi.dtype)

  piT = pi.T
  gT = g.T
  mu5 = mu.transpose(1, 2, 0).reshape(K, _NDT, 8, B // 128, 128)
  mu5 = mu5.transpose(0, 1, 3, 2, 4).reshape(K, _M, 8, 128)
  sg5 = sigma.transpose(1, 2, 0).reshape(K, _NDT, 8, B // 128, 128)
  sg5 = sg5.transpose(0, 1, 3, 2, 4).reshape(K, _M, 8, 128)

  omu = pl.pallas_call(
      _tc_body,
      grid=(B // 128 // _MB, _NDT),
      in_specs=[
          pl.BlockSpec((K, _TCW), lambda bt, dt: (0, bt)),
          pl.BlockSpec((K, _TCW), lambda bt, dt: (0, bt)),
          pl.BlockSpec((K, _MB, 8, 128),
                       lambda bt, dt: (0, dt * (_B // 128 // _MB) + bt, 0, 0)),
      ],
      out_specs=pl.BlockSpec((8, _TCW), lambda bt, dt: (dt, bt)),
      out_shape=jax.ShapeDtypeStruct((D, B), jnp.float32),
      scratch_shapes=[pltpu.VMEM((8, _TCW), jnp.int32)],
      compiler_params=pltpu.CompilerParams(
          dimension_semantics=("arbitrary", "arbitrary")),
  )(piT, gT, mu5)

  mesh = plsc.VectorSubcoreMesh(core_axis_name="c", subcore_axis_name="s")
  run = pl.kernel(
      _sc_body,
      out_type=(
          jax.ShapeDtypeStruct((D, B), jnp.float32),
          jax.ShapeDtypeStruct((B,), jnp.int32),
      ),
      mesh=mesh,
      compiler_params=pltpu.CompilerParams(needs_layout_passes=False),
      scratch_types=[
          pltpu.VMEM((K, _BPW), jnp.float32),
          pltpu.VMEM((K, _BPW), jnp.float32),
          pltpu.VMEM((_BPW,), jnp.int32),
          pltpu.VMEM((K, 8, 128), jnp.float32),
          pltpu.VMEM((K, 8, 128), jnp.float32),
          pltpu.VMEM((8, 128), jnp.float32),
          pltpu.SemaphoreType.DMA,
          pltpu.SemaphoreType.DMA,
      ],
  )
  osg, idx = run(piT, gT, sg5)
  return omu.T, osg.T, idx
